# baseline (device time: 36567 ns/iter reference)
import jax
import jax.numpy as jnp
from jax import lax
from jax.experimental import pallas as pl
from jax.experimental.pallas import tpu as pltpu

N = 16
H = 2


def kernel(x):
    m, n = x.shape
    ch = m // N
    hh = ch // H

    def body(x_ref, out_ref, rs_buf, rs_send, rs_recv, ag_send, ag_recv):
        p = lax.axis_index("i")

        rs_rdmas = [[] for _ in range(H)]
        for d in range(1, N):
            t = (p + d) % N
            rdma = pltpu.make_async_remote_copy(
                src_ref=x_ref.at[pl.ds(t * ch, hh), :],
                dst_ref=rs_buf.at[0, d],
                send_sem=rs_send.at[0, d],
                recv_sem=rs_recv.at[0, d],
                device_id=(t,),
                device_id_type=pl.DeviceIdType.MESH,
            )
            rdma.start()
            rs_rdmas[0].append(rdma)

        for h in range(H):
            rs_buf[h, 0, :, :] = x_ref[pl.ds(p * ch + h * hh, hh), :]

        ag_rdmas = []
        for h in range(H):
            if h + 1 < H:
                for rdma in rs_rdmas[h]:
                    rdma.wait_send()
                for d in range(1, N):
                    t = (p + d) % N
                    rdma = pltpu.make_async_remote_copy(
                        src_ref=x_ref.at[pl.ds(t * ch + (h + 1) * hh, hh), :],
                        dst_ref=rs_buf.at[h + 1, d],
                        send_sem=rs_send.at[h + 1, d],
                        recv_sem=rs_recv.at[h + 1, d],
                        device_id=(t,),
                        device_id_type=pl.DeviceIdType.MESH,
                    )
                    rdma.start()
                    rs_rdmas[h + 1].append(rdma)
            for rdma in rs_rdmas[h]:
                rdma.wait_recv()
            out_ref[pl.ds(p * ch + h * hh, hh), :] = jnp.sum(
                rs_buf[h], axis=0
            )
            for d in range(1, N):
                t = (p + d) % N
                rdma = pltpu.make_async_remote_copy(
                    src_ref=out_ref.at[pl.ds(p * ch + h * hh, hh), :],
                    dst_ref=out_ref.at[pl.ds(p * ch + h * hh, hh), :],
                    send_sem=ag_send.at[h, d],
                    recv_sem=ag_recv.at[h, d],
                    device_id=(t,),
                    device_id_type=pl.DeviceIdType.MESH,
                )
                rdma.start()
                ag_rdmas.append(rdma)

        for rdma in ag_rdmas:
            rdma.wait_recv()

        for rdma in rs_rdmas[H - 1]:
            rdma.wait_send()
        for rdma in ag_rdmas:
            rdma.wait_send()

    return pl.pallas_call(
        body,
        out_shape=jax.ShapeDtypeStruct((m, n), x.dtype),
        in_specs=[pl.BlockSpec(memory_space=pltpu.VMEM)],
        out_specs=pl.BlockSpec(memory_space=pltpu.VMEM),
        scratch_shapes=[
            pltpu.VMEM((H, N, hh, n), x.dtype),
            pltpu.SemaphoreType.DMA((H, N)),
            pltpu.SemaphoreType.DMA((H, N)),
            pltpu.SemaphoreType.DMA((H, N)),
            pltpu.SemaphoreType.DMA((H, N)),
        ],
    )(x)


# device time: 29712 ns/iter; 1.2307x vs baseline; 1.2307x over previous
import jax
import jax.numpy as jnp
from jax import lax
from jax.experimental import pallas as pl
from jax.experimental.pallas import tpu as pltpu

N = 16


def kernel(x):
    m, n = x.shape
    ch = m // N

    def body(x_ref, out_ref, rs_buf, rs_send, rs_recv, ag_send, ag_recv):
        p = lax.axis_index("i")

        barrier_sem = pltpu.get_barrier_semaphore()
        for d in range(1, N):
            pl.semaphore_signal(
                barrier_sem, inc=1,
                device_id=((p + d) % N,),
                device_id_type=pl.DeviceIdType.MESH,
            )
        pl.semaphore_wait(barrier_sem, N - 1)

        rs_rdmas = []
        for d in range(1, N):
            t = (p + d) % N
            rdma = pltpu.make_async_remote_copy(
                src_ref=x_ref.at[pl.ds(t * ch, ch), :],
                dst_ref=rs_buf.at[d],
                send_sem=rs_send.at[d],
                recv_sem=rs_recv.at[d],
                device_id=(t,),
                device_id_type=pl.DeviceIdType.MESH,
            )
            rdma.start()
            rs_rdmas.append(rdma)

        rs_buf[0, :, :] = x_ref[pl.ds(p * ch, ch), :]

        for rdma in rs_rdmas:
            rdma.wait_recv()
        out_ref[pl.ds(p * ch, ch), :] = jnp.sum(rs_buf[...], axis=0)

        ag_rdmas = []
        for d in range(1, N):
            t = (p + d) % N
            rdma = pltpu.make_async_remote_copy(
                src_ref=out_ref.at[pl.ds(p * ch, ch), :],
                dst_ref=out_ref.at[pl.ds(p * ch, ch), :],
                send_sem=ag_send.at[d],
                recv_sem=ag_recv.at[d],
                device_id=(t,),
                device_id_type=pl.DeviceIdType.MESH,
            )
            rdma.start()
            ag_rdmas.append(rdma)

        for rdma in ag_rdmas:
            rdma.wait_recv()

        for rdma in rs_rdmas:
            rdma.wait_send()
        for rdma in ag_rdmas:
            rdma.wait_send()

    return pl.pallas_call(
        body,
        out_shape=jax.ShapeDtypeStruct((m, n), x.dtype),
        in_specs=[pl.BlockSpec(memory_space=pltpu.VMEM)],
        out_specs=pl.BlockSpec(memory_space=pltpu.VMEM),
        scratch_shapes=[
            pltpu.VMEM((N, ch, n), x.dtype),
            pltpu.SemaphoreType.DMA((N,)),
            pltpu.SemaphoreType.DMA((N,)),
            pltpu.SemaphoreType.DMA((N,)),
            pltpu.SemaphoreType.DMA((N,)),
        ],
        compiler_params=pltpu.CompilerParams(collective_id=0),
    )(x)
